# baseline (speedup 1.0000x reference)
import functools

import jax
import jax.numpy as jnp
from jax import lax
from jax.experimental import pallas as pl
from jax.experimental.pallas import tpu as pltpu
from jax.experimental.pallas import tpu_sc as plsc

N = 100000
QN = N // 4
D_TOTAL = 128
D_DYN = 32
D_ANC = 96
HIDDEN = 64
NSTEPS = 4
DT = 0.25

NC = 2
NS = 16
NW = NC * NS
P_PER_W = N // NW
CHUNK = 125
NCH = P_PER_W // CHUNK

_HIGH = lax.Precision.HIGHEST


@functools.lru_cache(maxsize=None)
def _make_gather(nj, width):
  mesh = plsc.VectorSubcoreMesh(core_axis_name="c", subcore_axis_name="s")

  @functools.partial(
      pl.kernel,
      out_type=jax.ShapeDtypeStruct((N, nj * width), jnp.float32),
      mesh=mesh,
      compiler_params=pltpu.CompilerParams(use_tc_tiling_on_sc=False),
      scratch_types=[
          pltpu.VMEM((NCH, nj, CHUNK), jnp.int32),
          pltpu.VMEM((nj, CHUNK, width), jnp.float32),
          pltpu.SemaphoreType.DMA,
      ],
  )
  def gather_kernel(idx_hbm, table_hbm, out_hbm, idx_v, buf, sem):
    wid = lax.axis_index("s") * NC + lax.axis_index("c")
    pltpu.sync_copy(idx_hbm.at[wid], idx_v)

    def body(c, carry):
      copies = [
          pltpu.async_copy(table_hbm.at[idx_v.at[c, j]], buf.at[j], sem)
          for j in range(nj)
      ]
      for cp in copies:
        cp.wait()
      base = wid * P_PER_W + c * CHUNK
      for j in range(nj):
        pltpu.sync_copy(
            buf.at[j],
            out_hbm.at[pl.ds(base, CHUNK), pl.ds(j * width, width)],
        )
      return carry

    lax.fori_loop(0, NCH, body, 0)

  return gather_kernel


_BLK = 1000
_NBLK = QN // _BLK
_QB = QN // _BLK


def _quarter_specs(w):
  return [
      pl.BlockSpec((_BLK, w), functools.partial(lambda q, i: (q * _QB + i, 0), q))
      for q in range(4)
  ]


def _pre_body(a0, a1, a2, a3, g0, g1, g2, g3, wa_ref, wn_ref, b1_ref, out_ref):
  anc_q = (a0, a1, a2, a3)
  ganc_q = (g0, g1, g2, g3)
  for q in range(4):
    aq = (
        b1_ref[...]
        + jnp.dot(anc_q[q][...], wa_ref[...], precision=_HIGH)
        + jnp.dot(ganc_q[q][...], wn_ref[...], precision=_HIGH)
    )
    out_ref[:, q * HIDDEN:(q + 1) * HIDDEN] = aq


def _step_body(g0, g1, g2, g3, a_ref, wd_ref, w2_ref, b2_ref, out_ref):
  g_q = (g0, g1, g2, g3)
  for q in range(4):
    g4 = g_q[q][...]
    h = a_ref[:, q * HIDDEN:(q + 1) * HIDDEN] + jnp.dot(
        g4, wd_ref[...], precision=_HIGH)
    fz = jnp.dot(jax.nn.gelu(h), w2_ref[...], precision=_HIGH) + b2_ref[...]
    out_ref[:, q * D_DYN:(q + 1) * D_DYN] = g4[:, :D_DYN] + DT * fz


def _fold_spec(w):
  return pl.BlockSpec((_BLK, w), lambda i: (i, 0))


def _full_spec(r, c):
  return pl.BlockSpec((r, c), lambda i: (0, 0))


_precompute = pl.pallas_call(
    _pre_body,
    grid=(_NBLK,),
    in_specs=(
        _quarter_specs(D_ANC)
        + _quarter_specs(3 * D_ANC)
        + [
            _full_spec(D_ANC, HIDDEN),
            _full_spec(3 * D_ANC, HIDDEN),
            _full_spec(1, HIDDEN),
        ]
    ),
    out_specs=_fold_spec(4 * HIDDEN),
    out_shape=jax.ShapeDtypeStruct((QN, 4 * HIDDEN), jnp.float32),
)

_step = pl.pallas_call(
    _step_body,
    grid=(_NBLK,),
    in_specs=(
        _quarter_specs(4 * D_DYN)
        + [
            _fold_spec(4 * HIDDEN),
            _full_spec(4 * D_DYN, HIDDEN),
            _full_spec(HIDDEN, D_DYN),
            _full_spec(1, D_DYN),
        ]
    ),
    out_specs=_fold_spec(4 * D_DYN),
    out_shape=jax.ShapeDtypeStruct((QN, 4 * D_DYN), jnp.float32),
)


def kernel(x, neighbour_index, W1, b1, W2, b2):
  nb = neighbour_index.reshape(NW, NCH, CHUNK, 4)
  nbr = 4 * (nb % QN) + nb // QN
  idx4 = nbr.transpose(0, 1, 3, 2)
  idx3 = nb[..., 1:].transpose(0, 1, 3, 2)
  anc = x[:, D_DYN:]

  w1r = W1.reshape(4, D_TOTAL, HIDDEN)
  wd = w1r[:, :D_DYN].reshape(4 * D_DYN, HIDDEN)
  wa_self = w1r[0, D_DYN:]
  wa_nbr = w1r[1:, D_DYN:].reshape(3 * D_ANC, HIDDEN)

  ganc = _make_gather(3, D_ANC)(idx3, anc)
  a_fold = _precompute(
      anc, anc, anc, anc, ganc, ganc, ganc, ganc,
      wa_self, wa_nbr, b1.reshape(1, HIDDEN))

  table = jnp.concatenate(
      [x[q * QN:(q + 1) * QN, :D_DYN] for q in range(4)], axis=1)
  for _ in range(NSTEPS):
    g4 = _make_gather(4, D_DYN)(idx4, table.reshape(N, D_DYN))
    table = _step(g4, g4, g4, g4, a_fold, wd, W2, b2.reshape(1, D_DYN))

  dyn = jnp.concatenate(
      [table[:, q * D_DYN:(q + 1) * D_DYN] for q in range(4)], axis=0)
  return jnp.concatenate([dyn, anc], axis=1)


# speedup vs baseline: 1.8304x; 1.8304x over previous
import functools

import jax
import jax.numpy as jnp
from jax import lax
from jax.experimental import pallas as pl
from jax.experimental.pallas import tpu as pltpu
from jax.experimental.pallas import tpu_sc as plsc

N = 100000
QN = N // 4
D_TOTAL = 128
D_DYN = 32
D_ANC = 96
HIDDEN = 64
NSTEPS = 4
DT = 0.25

NC = 2
NS = 16
NW = NC * NS
P_PER_W = N // NW
CHUNK = 125
NCH = P_PER_W // CHUNK

_HIGH = lax.Precision.DEFAULT


@functools.lru_cache(maxsize=None)
def _make_gather(nj, width):
  mesh = plsc.VectorSubcoreMesh(core_axis_name="c", subcore_axis_name="s")

  @functools.partial(
      pl.kernel,
      out_type=jax.ShapeDtypeStruct((N, nj * width), jnp.float32),
      mesh=mesh,
      compiler_params=pltpu.CompilerParams(use_tc_tiling_on_sc=False),
      scratch_types=[
          pltpu.VMEM((NCH, nj, CHUNK), jnp.int32),
          pltpu.VMEM((nj, CHUNK, width), jnp.float32),
          pltpu.SemaphoreType.DMA,
      ],
  )
  def gather_kernel(idx_hbm, table_hbm, out_hbm, idx_v, buf, sem):
    wid = lax.axis_index("s") * NC + lax.axis_index("c")
    pltpu.sync_copy(idx_hbm.at[wid], idx_v)

    def body(c, carry):
      copies = [
          pltpu.async_copy(table_hbm.at[idx_v.at[c, j]], buf.at[j], sem)
          for j in range(nj)
      ]
      for cp in copies:
        cp.wait()
      base = wid * P_PER_W + c * CHUNK
      for j in range(nj):
        pltpu.sync_copy(
            buf.at[j],
            out_hbm.at[pl.ds(base, CHUNK), pl.ds(j * width, width)],
        )
      return carry

    lax.fori_loop(0, NCH, body, 0)

  return gather_kernel


_BLK = 1000
_NBLK = QN // _BLK
_QB = QN // _BLK


def _quarter_specs(w):
  return [
      pl.BlockSpec((_BLK, w), functools.partial(lambda q, i: (q * _QB + i, 0), q))
      for q in range(4)
  ]


def _pre_body(a0, a1, a2, a3, g0, g1, g2, g3, wa_ref, wn_ref, b1_ref, out_ref):
  anc_q = (a0, a1, a2, a3)
  ganc_q = (g0, g1, g2, g3)
  for q in range(4):
    aq = (
        b1_ref[...]
        + jnp.dot(anc_q[q][...], wa_ref[...], precision=_HIGH)
        + jnp.dot(ganc_q[q][...], wn_ref[...], precision=_HIGH)
    )
    out_ref[:, q * HIDDEN:(q + 1) * HIDDEN] = aq


def _step_body(g0, g1, g2, g3, a_ref, wd_ref, w2_ref, b2_ref, out_ref):
  g_q = (g0, g1, g2, g3)
  for q in range(4):
    g4 = g_q[q][...]
    h = a_ref[:, q * HIDDEN:(q + 1) * HIDDEN] + jnp.dot(
        g4, wd_ref[...], precision=_HIGH)
    fz = jnp.dot(jax.nn.gelu(h), w2_ref[...], precision=_HIGH) + b2_ref[...]
    out_ref[:, q * D_DYN:(q + 1) * D_DYN] = g4[:, :D_DYN] + DT * fz


def _fold_spec(w):
  return pl.BlockSpec((_BLK, w), lambda i: (i, 0))


def _full_spec(r, c):
  return pl.BlockSpec((r, c), lambda i: (0, 0))


_precompute = pl.pallas_call(
    _pre_body,
    grid=(_NBLK,),
    in_specs=(
        _quarter_specs(D_ANC)
        + _quarter_specs(3 * D_ANC)
        + [
            _full_spec(D_ANC, HIDDEN),
            _full_spec(3 * D_ANC, HIDDEN),
            _full_spec(1, HIDDEN),
        ]
    ),
    out_specs=_fold_spec(4 * HIDDEN),
    out_shape=jax.ShapeDtypeStruct((QN, 4 * HIDDEN), jnp.float32),
)

_step = pl.pallas_call(
    _step_body,
    grid=(_NBLK,),
    in_specs=(
        _quarter_specs(4 * D_DYN)
        + [
            _fold_spec(4 * HIDDEN),
            _full_spec(4 * D_DYN, HIDDEN),
            _full_spec(HIDDEN, D_DYN),
            _full_spec(1, D_DYN),
        ]
    ),
    out_specs=_fold_spec(4 * D_DYN),
    out_shape=jax.ShapeDtypeStruct((QN, 4 * D_DYN), jnp.float32),
)


def kernel(x, neighbour_index, W1, b1, W2, b2):
  nb = neighbour_index.reshape(NW, NCH, CHUNK, 4)
  nbr = 4 * (nb % QN) + nb // QN
  idx4 = nbr.transpose(0, 1, 3, 2)
  idx3 = nb[..., 1:].transpose(0, 1, 3, 2)
  anc = x[:, D_DYN:]

  w1r = W1.reshape(4, D_TOTAL, HIDDEN)
  wd = w1r[:, :D_DYN].reshape(4 * D_DYN, HIDDEN)
  wa_self = w1r[0, D_DYN:]
  wa_nbr = w1r[1:, D_DYN:].reshape(3 * D_ANC, HIDDEN)

  ganc = _make_gather(3, D_ANC)(idx3, anc)
  a_fold = _precompute(
      anc, anc, anc, anc, ganc, ganc, ganc, ganc,
      wa_self, wa_nbr, b1.reshape(1, HIDDEN))

  table = jnp.concatenate(
      [x[q * QN:(q + 1) * QN, :D_DYN] for q in range(4)], axis=1)
  for _ in range(NSTEPS):
    g4 = _make_gather(4, D_DYN)(idx4, table.reshape(N, D_DYN))
    table = _step(g4, g4, g4, g4, a_fold, wd, W2, b2.reshape(1, D_DYN))

  dyn = jnp.concatenate(
      [table[:, q * D_DYN:(q + 1) * D_DYN] for q in range(4)], axis=0)
  return jnp.concatenate([dyn, anc], axis=1)


# speedup vs baseline: 2.5215x; 1.3776x over previous
import functools

import jax
import jax.numpy as jnp
from jax import lax
from jax.experimental import pallas as pl
from jax.experimental.pallas import tpu as pltpu
from jax.experimental.pallas import tpu_sc as plsc

N = 100000
QN = N // 4
D_TOTAL = 128
D_DYN = 32
D_ANC = 96
HIDDEN = 64
NSTEPS = 4
DT = 0.25

NC = 2
NS = 16
NW = NC * NS
P_PER_W = N // NW
CHUNK = 125
NCH = P_PER_W // CHUNK


@functools.lru_cache(maxsize=None)
def _make_gather(nj, width, planes):
  mesh = plsc.VectorSubcoreMesh(core_axis_name="c", subcore_axis_name="s")
  out_shape = (nj, N, width) if planes else (N, nj * width)

  @functools.partial(
      pl.kernel,
      out_type=jax.ShapeDtypeStruct(out_shape, jnp.float32),
      mesh=mesh,
      compiler_params=pltpu.CompilerParams(use_tc_tiling_on_sc=False),
      scratch_types=[
          pltpu.VMEM((NCH, nj, CHUNK), jnp.int32),
          pltpu.VMEM((nj, CHUNK, width), jnp.float32),
          pltpu.SemaphoreType.DMA,
      ],
  )
  def gather_kernel(idx_hbm, table_hbm, out_hbm, idx_v, buf, sem):
    wid = lax.axis_index("s") * NC + lax.axis_index("c")
    pltpu.sync_copy(idx_hbm.at[wid], idx_v)

    def body(c, carry):
      copies = [
          pltpu.async_copy(table_hbm.at[idx_v.at[c, j]], buf.at[j], sem)
          for j in range(nj)
      ]
      for cp in copies:
        cp.wait()
      base = wid * P_PER_W + c * CHUNK
      for j in range(nj):
        if planes:
          dst = out_hbm.at[j, pl.ds(base, CHUNK)]
        else:
          dst = out_hbm.at[pl.ds(base, CHUNK), pl.ds(j * width, width)]
        pltpu.sync_copy(buf.at[j], dst)
      return carry

    lax.fori_loop(0, NCH, body, 0)

  return gather_kernel


_BLK = 1000
_NBLK = QN // _BLK
_QB = QN // _BLK


def _quarter_specs(w):
  return [
      pl.BlockSpec((_BLK, w), functools.partial(lambda q, i: (q * _QB + i, 0), q))
      for q in range(4)
  ]


def _plane_specs():
  return [
      pl.BlockSpec(
          (1, _BLK, D_TOTAL),
          functools.partial(lambda j, q, i: (j, q * _QB + i, 0), j, q),
      )
      for j in range(3)
      for q in range(4)
  ]


def _pre0_body(x0, x1, x2, x3,
               g00, g01, g02, g03, g10, g11, g12, g13, g20, g21, g22, g23,
               wa_ref, wn_ref, wd_ref, w2_ref, b1_ref, b2_ref,
               a_out, t_out):
  x_q = (x0, x1, x2, x3)
  xg = ((g00, g01, g02, g03), (g10, g11, g12, g13), (g20, g21, g22, g23))
  for q in range(4):
    xq = x_q[q][...]
    az = b1_ref[...] + jnp.dot(xq[:, D_DYN:], wa_ref[...])
    h = jnp.dot(xq[:, :D_DYN], wd_ref[:D_DYN])
    for j in range(3):
      gj = xg[j][q][0]
      az = az + jnp.dot(gj[:, D_DYN:], wn_ref[j])
      h = h + jnp.dot(gj[:, :D_DYN], wd_ref[(j + 1) * D_DYN:(j + 2) * D_DYN])
    h = h + az
    fz = jnp.dot(jax.nn.gelu(h), w2_ref[...]) + b2_ref[...]
    a_out[:, q * HIDDEN:(q + 1) * HIDDEN] = az
    t_out[:, q * D_DYN:(q + 1) * D_DYN] = xq[:, :D_DYN] + DT * fz


def _step_body(g0, g1, g2, g3, a_ref, wd_ref, w2_ref, b2_ref, out_ref):
  g_q = (g0, g1, g2, g3)
  for q in range(4):
    g4 = g_q[q][...]
    h = a_ref[:, q * HIDDEN:(q + 1) * HIDDEN] + jnp.dot(g4, wd_ref[...])
    fz = jnp.dot(jax.nn.gelu(h), w2_ref[...]) + b2_ref[...]
    out_ref[:, q * D_DYN:(q + 1) * D_DYN] = g4[:, :D_DYN] + DT * fz


def _final_body(g4_ref, x_ref, a_ref, wd_ref, w2_ref, b2_ref, out_ref):
  q = pl.program_id(0) // _QB
  a_all = a_ref[...]
  az = jnp.where(
      q == 0, a_all[:, 0 * HIDDEN:1 * HIDDEN],
      jnp.where(
          q == 1, a_all[:, 1 * HIDDEN:2 * HIDDEN],
          jnp.where(q == 2, a_all[:, 2 * HIDDEN:3 * HIDDEN],
                    a_all[:, 3 * HIDDEN:4 * HIDDEN])))
  g4 = g4_ref[...]
  h = az + jnp.dot(g4, wd_ref[...])
  fz = jnp.dot(jax.nn.gelu(h), w2_ref[...]) + b2_ref[...]
  out_ref[:, :D_DYN] = g4[:, :D_DYN] + DT * fz
  out_ref[:, D_DYN:] = x_ref[:, D_DYN:]


def _fold_spec(w):
  return pl.BlockSpec((_BLK, w), lambda i: (i, 0))


def _full_spec(*shape):
  n = len(shape)
  return pl.BlockSpec(shape, lambda i: (0,) * n)


_pre0 = pl.pallas_call(
    _pre0_body,
    grid=(_NBLK,),
    in_specs=(
        _quarter_specs(D_TOTAL)
        + _plane_specs()
        + [
            _full_spec(D_ANC, HIDDEN),
            _full_spec(3, D_ANC, HIDDEN),
            _full_spec(4 * D_DYN, HIDDEN),
            _full_spec(HIDDEN, D_DYN),
            _full_spec(1, HIDDEN),
            _full_spec(1, D_DYN),
        ]
    ),
    out_specs=[_fold_spec(4 * HIDDEN), _fold_spec(4 * D_DYN)],
    out_shape=[
        jax.ShapeDtypeStruct((QN, 4 * HIDDEN), jnp.float32),
        jax.ShapeDtypeStruct((QN, 4 * D_DYN), jnp.float32),
    ],
)

_step = pl.pallas_call(
    _step_body,
    grid=(_NBLK,),
    in_specs=(
        _quarter_specs(4 * D_DYN)
        + [
            _fold_spec(4 * HIDDEN),
            _full_spec(4 * D_DYN, HIDDEN),
            _full_spec(HIDDEN, D_DYN),
            _full_spec(1, D_DYN),
        ]
    ),
    out_specs=_fold_spec(4 * D_DYN),
    out_shape=jax.ShapeDtypeStruct((QN, 4 * D_DYN), jnp.float32),
)

_final = pl.pallas_call(
    _final_body,
    grid=(4 * _NBLK,),
    in_specs=[
        pl.BlockSpec((_BLK, D_TOTAL), lambda i: (i, 0)),
        pl.BlockSpec((_BLK, D_TOTAL), lambda i: (i, 0)),
        pl.BlockSpec((_BLK, 4 * HIDDEN), lambda i: (i % _QB, 0)),
        _full_spec(4 * D_DYN, HIDDEN),
        _full_spec(HIDDEN, D_DYN),
        _full_spec(1, D_DYN),
    ],
    out_specs=pl.BlockSpec((_BLK, D_TOTAL), lambda i: (i, 0)),
    out_shape=jax.ShapeDtypeStruct((N, D_TOTAL), jnp.float32),
)


def kernel(x, neighbour_index, W1, b1, W2, b2):
  nb = neighbour_index.reshape(NW, NCH, CHUNK, 4)
  nbr = 4 * (nb % QN) + nb // QN
  idx4 = nbr.transpose(0, 1, 3, 2)
  idx3 = nb[..., 1:].transpose(0, 1, 3, 2)

  w1r = W1.reshape(4, D_TOTAL, HIDDEN)
  wd = w1r[:, :D_DYN].reshape(4 * D_DYN, HIDDEN)
  wa_self = w1r[0, D_DYN:]
  wa_nbr = w1r[1:, D_DYN:]

  xg = _make_gather(3, D_TOTAL, True)(idx3, x)
  a_fold, table = _pre0(
      x, x, x, x,
      xg, xg, xg, xg, xg, xg, xg, xg, xg, xg, xg, xg,
      wa_self, wa_nbr, wd, W2, b1.reshape(1, HIDDEN), b2.reshape(1, D_DYN))

  b2r = b2.reshape(1, D_DYN)
  for _ in range(NSTEPS - 2):
    g4 = _make_gather(4, D_DYN, False)(idx4, table.reshape(N, D_DYN))
    table = _step(g4, g4, g4, g4, a_fold, wd, W2, b2r)

  g4 = _make_gather(4, D_DYN, False)(idx4, table.reshape(N, D_DYN))
  return _final(g4, x, a_fold, wd, W2, b2r)


# speedup vs baseline: 2.8153x; 1.1165x over previous
import functools

import jax
import jax.numpy as jnp
from jax import lax
from jax.experimental import pallas as pl
from jax.experimental.pallas import tpu as pltpu
from jax.experimental.pallas import tpu_sc as plsc

N = 100000
QN = N // 4
D_TOTAL = 128
D_DYN = 32
D_ANC = 96
HIDDEN = 64
NSTEPS = 4
DT = 0.25

NC = 2
NS = 16
NW = NC * NS
P_PER_W = N // NW
CHUNK = 125
NCH = P_PER_W // CHUNK


@functools.lru_cache(maxsize=None)
def _make_gather(nj, width, planes):
  mesh = plsc.VectorSubcoreMesh(core_axis_name="c", subcore_axis_name="s")
  out_shape = (nj, N, width) if planes else (N, nj * width)

  @functools.partial(
      pl.kernel,
      out_type=jax.ShapeDtypeStruct(out_shape, jnp.float32),
      mesh=mesh,
      compiler_params=pltpu.CompilerParams(use_tc_tiling_on_sc=False),
      scratch_types=[
          pltpu.VMEM((NCH, nj, CHUNK), jnp.int32),
          pltpu.VMEM((nj, CHUNK, width), jnp.float32),
          pltpu.SemaphoreType.DMA,
      ],
  )
  def gather_kernel(idx_hbm, table_hbm, out_hbm, idx_v, buf, sem):
    wid = lax.axis_index("s") * NC + lax.axis_index("c")
    pltpu.sync_copy(idx_hbm.at[wid], idx_v)

    def body(c, carry):
      copies = [
          pltpu.async_copy(table_hbm.at[idx_v.at[c, j]], buf.at[j], sem)
          for j in range(nj)
      ]
      for cp in copies:
        cp.wait()
      base = wid * P_PER_W + c * CHUNK
      for j in range(nj):
        if planes:
          dst = out_hbm.at[j, pl.ds(base, CHUNK)]
        else:
          dst = out_hbm.at[pl.ds(base, CHUNK), pl.ds(j * width, width)]
        pltpu.sync_copy(buf.at[j], dst)
      return carry

    lax.fori_loop(0, NCH, body, 0)

  return gather_kernel


_BLK = 1000
_NBLK = QN // _BLK
_QB = QN // _BLK
_SBLK = 5000
_SNB = QN // _SBLK
_SQB = QN // _SBLK


def _quarter_specs(w, blk=_BLK):
  qb = QN // blk
  return [
      pl.BlockSpec((blk, w), functools.partial(lambda q, i: (q * qb + i, 0), q))
      for q in range(4)
  ]


def _plane_specs():
  return [
      pl.BlockSpec(
          (1, _BLK, D_TOTAL),
          functools.partial(lambda j, q, i: (j, q * _QB + i, 0), j, q),
      )
      for j in range(3)
      for q in range(4)
  ]


def _pre0_body(x0, x1, x2, x3,
               g00, g01, g02, g03, g10, g11, g12, g13, g20, g21, g22, g23,
               wa_ref, wn_ref, wd_ref, w2_ref, b1_ref, b2_ref,
               a_out, t_out):
  x_q = (x0, x1, x2, x3)
  xg = ((g00, g01, g02, g03), (g10, g11, g12, g13), (g20, g21, g22, g23))
  for q in range(4):
    xq = x_q[q][...]
    az = b1_ref[...] + jnp.dot(xq[:, D_DYN:], wa_ref[...])
    h = jnp.dot(xq[:, :D_DYN], wd_ref[:D_DYN])
    for j in range(3):
      gj = xg[j][q][0]
      az = az + jnp.dot(gj[:, D_DYN:], wn_ref[j])
      h = h + jnp.dot(gj[:, :D_DYN], wd_ref[(j + 1) * D_DYN:(j + 2) * D_DYN])
    h = h + az
    fz = jnp.dot(jax.nn.gelu(h), w2_ref[...]) + b2_ref[...]
    a_out[:, q * HIDDEN:(q + 1) * HIDDEN] = az
    t_out[:, q * D_DYN:(q + 1) * D_DYN] = xq[:, :D_DYN] + DT * fz


def _step_body(g0, g1, g2, g3, a_ref, wd_ref, w2_ref, b2_ref, out_ref):
  g_q = (g0, g1, g2, g3)
  for q in range(4):
    g4 = g_q[q][...]
    h = a_ref[:, q * HIDDEN:(q + 1) * HIDDEN] + jnp.dot(g4, wd_ref[...])
    fz = jnp.dot(jax.nn.gelu(h), w2_ref[...]) + b2_ref[...]
    out_ref[:, q * D_DYN:(q + 1) * D_DYN] = g4[:, :D_DYN] + DT * fz


def _final_body(g4_ref, x_ref, a_ref, wd_ref, w2_ref, b2_ref, out_ref):
  q = pl.program_id(1)
  a_all = a_ref[...]
  az = jnp.where(
      q == 0, a_all[:, 0 * HIDDEN:1 * HIDDEN],
      jnp.where(
          q == 1, a_all[:, 1 * HIDDEN:2 * HIDDEN],
          jnp.where(q == 2, a_all[:, 2 * HIDDEN:3 * HIDDEN],
                    a_all[:, 3 * HIDDEN:4 * HIDDEN])))
  g4 = g4_ref[...]
  h = az + jnp.dot(g4, wd_ref[...])
  fz = jnp.dot(jax.nn.gelu(h), w2_ref[...]) + b2_ref[...]
  out_ref[:, :D_DYN] = g4[:, :D_DYN] + DT * fz
  out_ref[:, D_DYN:] = x_ref[:, D_DYN:]


def _fold_spec(w, blk=_BLK):
  return pl.BlockSpec((blk, w), lambda i: (i, 0))


def _full_spec(*shape):
  n = len(shape)
  return pl.BlockSpec(shape, lambda *_: (0,) * n)


_pre0 = pl.pallas_call(
    _pre0_body,
    grid=(_NBLK,),
    in_specs=(
        _quarter_specs(D_TOTAL)
        + _plane_specs()
        + [
            _full_spec(D_ANC, HIDDEN),
            _full_spec(3, D_ANC, HIDDEN),
            _full_spec(4 * D_DYN, HIDDEN),
            _full_spec(HIDDEN, D_DYN),
            _full_spec(1, HIDDEN),
            _full_spec(1, D_DYN),
        ]
    ),
    out_specs=[_fold_spec(4 * HIDDEN), _fold_spec(4 * D_DYN)],
    out_shape=[
        jax.ShapeDtypeStruct((QN, 4 * HIDDEN), jnp.float32),
        jax.ShapeDtypeStruct((QN, 4 * D_DYN), jnp.float32),
    ],
)

_step = pl.pallas_call(
    _step_body,
    grid=(_SNB,),
    in_specs=(
        _quarter_specs(4 * D_DYN, _SBLK)
        + [
            _fold_spec(4 * HIDDEN, _SBLK),
            _full_spec(4 * D_DYN, HIDDEN),
            _full_spec(HIDDEN, D_DYN),
            _full_spec(1, D_DYN),
        ]
    ),
    out_specs=_fold_spec(4 * D_DYN, _SBLK),
    out_shape=jax.ShapeDtypeStruct((QN, 4 * D_DYN), jnp.float32),
)

_final = pl.pallas_call(
    _final_body,
    grid=(_SNB, 4),
    in_specs=[
        pl.BlockSpec((_SBLK, D_TOTAL), lambda a, q: (q * _SQB + a, 0)),
        pl.BlockSpec((_SBLK, D_TOTAL), lambda a, q: (q * _SQB + a, 0)),
        pl.BlockSpec((_SBLK, 4 * HIDDEN), lambda a, q: (a, 0)),
        _full_spec(4 * D_DYN, HIDDEN),
        _full_spec(HIDDEN, D_DYN),
        _full_spec(1, D_DYN),
    ],
    out_specs=pl.BlockSpec((_SBLK, D_TOTAL), lambda a, q: (q * _SQB + a, 0)),
    out_shape=jax.ShapeDtypeStruct((N, D_TOTAL), jnp.float32),
)


def kernel(x, neighbour_index, W1, b1, W2, b2):
  nb = neighbour_index.reshape(NW, NCH, CHUNK, 4)
  nbr = 4 * (nb % QN) + nb // QN
  idx4 = nbr.transpose(0, 1, 3, 2)
  idx3 = nb[..., 1:].transpose(0, 1, 3, 2)

  w1r = W1.reshape(4, D_TOTAL, HIDDEN)
  wd = w1r[:, :D_DYN].reshape(4 * D_DYN, HIDDEN)
  wa_self = w1r[0, D_DYN:]
  wa_nbr = w1r[1:, D_DYN:]

  xg = _make_gather(3, D_TOTAL, True)(idx3, x)
  a_fold, table = _pre0(
      x, x, x, x,
      xg, xg, xg, xg, xg, xg, xg, xg, xg, xg, xg, xg,
      wa_self, wa_nbr, wd, W2, b1.reshape(1, HIDDEN), b2.reshape(1, D_DYN))

  b2r = b2.reshape(1, D_DYN)
  for _ in range(NSTEPS - 2):
    g4 = _make_gather(4, D_DYN, False)(idx4, table.reshape(N, D_DYN))
    table = _step(g4, g4, g4, g4, a_fold, wd, W2, b2r)

  g4 = _make_gather(4, D_DYN, False)(idx4, table.reshape(N, D_DYN))
  return _final(g4, x, a_fold, wd, W2, b2r)


# speedup vs baseline: 3.1618x; 1.1231x over previous
import functools

import jax
import jax.numpy as jnp
from jax import lax
from jax.experimental import pallas as pl
from jax.experimental.pallas import tpu as pltpu
from jax.experimental.pallas import tpu_sc as plsc

N = 100000
QN = N // 4
D_TOTAL = 128
D_DYN = 32
D_ANC = 96
HIDDEN = 64
NSTEPS = 4
DT = 0.25

NC = 2
NS = 16
NW = NC * NS
P_PER_W = N // NW
CHUNK = 125
NCH = P_PER_W // CHUNK


@functools.lru_cache(maxsize=None)
def _make_gather(nj, width, planes):
  mesh = plsc.VectorSubcoreMesh(core_axis_name="c", subcore_axis_name="s")
  out_shape = (nj, N, width) if planes else (N, nj * width)

  @functools.partial(
      pl.kernel,
      out_type=jax.ShapeDtypeStruct(out_shape, jnp.float32),
      mesh=mesh,
      compiler_params=pltpu.CompilerParams(use_tc_tiling_on_sc=False),
      scratch_types=[
          pltpu.VMEM((NCH, nj, CHUNK), jnp.int32),
          pltpu.VMEM((nj, CHUNK, width), jnp.float32),
          pltpu.VMEM((nj, CHUNK, width), jnp.float32),
          pltpu.SemaphoreType.DMA,
          pltpu.SemaphoreType.DMA,
      ],
  )
  def gather_kernel(idx_hbm, table_hbm, out_hbm, idx_v, buf_a, buf_b, sem_a,
                    sem_b):
    wid = lax.axis_index("s") * NC + lax.axis_index("c")
    pltpu.sync_copy(idx_hbm.at[wid], idx_v)

    def fire(c, buf, sem):
      for j in range(nj):
        pltpu.async_copy(table_hbm.at[idx_v.at[c, j]], buf.at[j], sem)

    def drain_write(c, buf, sem):
      for j in range(nj):
        pltpu.make_async_copy(table_hbm.at[idx_v.at[c, j]], buf.at[j],
                              sem).wait()
      base = wid * P_PER_W + c * CHUNK
      for j in range(nj):
        if planes:
          dst = out_hbm.at[j, pl.ds(base, CHUNK)]
        else:
          dst = out_hbm.at[pl.ds(base, CHUNK), pl.ds(j * width, width)]
        pltpu.sync_copy(buf.at[j], dst)

    fire(0, buf_a, sem_a)

    def body(g, carry):
      c = 2 * g
      fire(c + 1, buf_b, sem_b)
      drain_write(c, buf_a, sem_a)
      fire(c + 2, buf_a, sem_a)
      drain_write(c + 1, buf_b, sem_b)
      return carry

    lax.fori_loop(0, (NCH - 1) // 2, body, 0)
    drain_write(NCH - 1, buf_a, sem_a)

  return gather_kernel


_BLK = 1000
_NBLK = QN // _BLK
_QB = QN // _BLK
_SBLK = 5000
_SNB = QN // _SBLK
_SQB = QN // _SBLK


def _quarter_specs(w, blk=_BLK):
  qb = QN // blk
  return [
      pl.BlockSpec((blk, w), functools.partial(lambda q, i: (q * qb + i, 0), q))
      for q in range(4)
  ]


def _plane_specs():
  return [
      pl.BlockSpec(
          (1, _BLK, D_TOTAL),
          functools.partial(lambda j, q, i: (j, q * _QB + i, 0), j, q),
      )
      for j in range(3)
      for q in range(4)
  ]


def _pre0_body(x0, x1, x2, x3,
               g00, g01, g02, g03, g10, g11, g12, g13, g20, g21, g22, g23,
               wa_ref, wn_ref, wd_ref, w2_ref, b1_ref, b2_ref,
               a_out, t_out):
  x_q = (x0, x1, x2, x3)
  xg = ((g00, g01, g02, g03), (g10, g11, g12, g13), (g20, g21, g22, g23))
  for q in range(4):
    xq = x_q[q][...]
    az = b1_ref[...] + jnp.dot(xq[:, D_DYN:], wa_ref[...])
    h = jnp.dot(xq[:, :D_DYN], wd_ref[:D_DYN])
    for j in range(3):
      gj = xg[j][q][0]
      az = az + jnp.dot(gj[:, D_DYN:], wn_ref[j])
      h = h + jnp.dot(gj[:, :D_DYN], wd_ref[(j + 1) * D_DYN:(j + 2) * D_DYN])
    h = h + az
    fz = jnp.dot(jax.nn.gelu(h), w2_ref[...]) + b2_ref[...]
    a_out[:, q * HIDDEN:(q + 1) * HIDDEN] = az
    t_out[:, q * D_DYN:(q + 1) * D_DYN] = xq[:, :D_DYN] + DT * fz


def _step_body(g0, g1, g2, g3, a_ref, wd_ref, w2_ref, b2_ref, out_ref):
  g_q = (g0, g1, g2, g3)
  for q in range(4):
    g4 = g_q[q][...]
    h = a_ref[:, q * HIDDEN:(q + 1) * HIDDEN] + jnp.dot(g4, wd_ref[...])
    fz = jnp.dot(jax.nn.gelu(h), w2_ref[...]) + b2_ref[...]
    out_ref[:, q * D_DYN:(q + 1) * D_DYN] = g4[:, :D_DYN] + DT * fz


def _final_body(g4_ref, x_ref, a_ref, wd_ref, w2_ref, b2_ref, out_ref):
  q = pl.program_id(1)
  a_all = a_ref[...]
  az = jnp.where(
      q == 0, a_all[:, 0 * HIDDEN:1 * HIDDEN],
      jnp.where(
          q == 1, a_all[:, 1 * HIDDEN:2 * HIDDEN],
          jnp.where(q == 2, a_all[:, 2 * HIDDEN:3 * HIDDEN],
                    a_all[:, 3 * HIDDEN:4 * HIDDEN])))
  g4 = g4_ref[...]
  h = az + jnp.dot(g4, wd_ref[...])
  fz = jnp.dot(jax.nn.gelu(h), w2_ref[...]) + b2_ref[...]
  out_ref[:, :D_DYN] = g4[:, :D_DYN] + DT * fz
  out_ref[:, D_DYN:] = x_ref[:, D_DYN:]


def _fold_spec(w, blk=_BLK):
  return pl.BlockSpec((blk, w), lambda i: (i, 0))


def _full_spec(*shape):
  n = len(shape)
  return pl.BlockSpec(shape, lambda *_: (0,) * n)


_pre0 = pl.pallas_call(
    _pre0_body,
    grid=(_NBLK,),
    in_specs=(
        _quarter_specs(D_TOTAL)
        + _plane_specs()
        + [
            _full_spec(D_ANC, HIDDEN),
            _full_spec(3, D_ANC, HIDDEN),
            _full_spec(4 * D_DYN, HIDDEN),
            _full_spec(HIDDEN, D_DYN),
            _full_spec(1, HIDDEN),
            _full_spec(1, D_DYN),
        ]
    ),
    out_specs=[_fold_spec(4 * HIDDEN), _fold_spec(4 * D_DYN)],
    out_shape=[
        jax.ShapeDtypeStruct((QN, 4 * HIDDEN), jnp.float32),
        jax.ShapeDtypeStruct((QN, 4 * D_DYN), jnp.float32),
    ],
)

_step = pl.pallas_call(
    _step_body,
    grid=(_SNB,),
    in_specs=(
        _quarter_specs(4 * D_DYN, _SBLK)
        + [
            _fold_spec(4 * HIDDEN, _SBLK),
            _full_spec(4 * D_DYN, HIDDEN),
            _full_spec(HIDDEN, D_DYN),
            _full_spec(1, D_DYN),
        ]
    ),
    out_specs=_fold_spec(4 * D_DYN, _SBLK),
    out_shape=jax.ShapeDtypeStruct((QN, 4 * D_DYN), jnp.float32),
)

_final = pl.pallas_call(
    _final_body,
    grid=(_SNB, 4),
    in_specs=[
        pl.BlockSpec((_SBLK, D_TOTAL), lambda a, q: (q * _SQB + a, 0)),
        pl.BlockSpec((_SBLK, D_TOTAL), lambda a, q: (q * _SQB + a, 0)),
        pl.BlockSpec((_SBLK, 4 * HIDDEN), lambda a, q: (a, 0)),
        _full_spec(4 * D_DYN, HIDDEN),
        _full_spec(HIDDEN, D_DYN),
        _full_spec(1, D_DYN),
    ],
    out_specs=pl.BlockSpec((_SBLK, D_TOTAL), lambda a, q: (q * _SQB + a, 0)),
    out_shape=jax.ShapeDtypeStruct((N, D_TOTAL), jnp.float32),
)


def kernel(x, neighbour_index, W1, b1, W2, b2):
  nb = neighbour_index.reshape(NW, NCH, CHUNK, 4)
  nbr = 4 * (nb % QN) + nb // QN
  idx4 = nbr.transpose(0, 1, 3, 2)
  idx3 = nb[..., 1:].transpose(0, 1, 3, 2)

  w1r = W1.reshape(4, D_TOTAL, HIDDEN)
  wd = w1r[:, :D_DYN].reshape(4 * D_DYN, HIDDEN)
  wa_self = w1r[0, D_DYN:]
  wa_nbr = w1r[1:, D_DYN:]

  xg = _make_gather(3, D_TOTAL, True)(idx3, x)
  a_fold, table = _pre0(
      x, x, x, x,
      xg, xg, xg, xg, xg, xg, xg, xg, xg, xg, xg, xg,
      wa_self, wa_nbr, wd, W2, b1.reshape(1, HIDDEN), b2.reshape(1, D_DYN))

  b2r = b2.reshape(1, D_DYN)
  for _ in range(NSTEPS - 2):
    g4 = _make_gather(4, D_DYN, False)(idx4, table.reshape(N, D_DYN))
    table = _step(g4, g4, g4, g4, a_fold, wd, W2, b2r)

  g4 = _make_gather(4, D_DYN, False)(idx4, table.reshape(N, D_DYN))
  return _final(g4, x, a_fold, wd, W2, b2r)
